# Initial kernel scaffold; baseline (speedup 1.0000x reference)
import jax
import jax.numpy as jnp
from jax.experimental import pallas as pl


def kernel(unknown, known, unknow_feats, known_feats, W1, b1, g1, be1, W2, b2, g2, be2):
    raise NotImplementedError("write your pallas kernel here")


# speedup vs baseline: 25.8295x; 25.8295x over previous
import functools

import jax
import jax.numpy as jnp
from jax.experimental import pallas as pl
from jax.experimental.pallas import tpu as pltpu

_EPS_D = 1e-8
_EPS_BN = 1e-5


def _kf1_kernel(w1a_ref, kf_ref, out_ref):
    w = w1a_ref[...].astype(jnp.bfloat16)
    f = kf_ref[0].astype(jnp.bfloat16)
    out_ref[0] = jnp.dot(w, f, preferred_element_type=jnp.float32).astype(
        jnp.bfloat16)


def _phase_a_kernel(ut_ref, kn_ref, kf1_ref, uf_ref, w1b_ref, b1_ref, g1_ref,
                    be1_ref, y1_ref, sc1_ref, sh1_ref, sum_ref, ssq_ref, *,
                    nt, m, cnt):
    ut = ut_ref[0]
    kn = kn_ref[0]
    u2 = ut[0:1, :] * ut[0:1, :] + ut[1:2, :] * ut[1:2, :] \
        + ut[2:3, :] * ut[2:3, :]
    k2 = kn[:, 0:1] * kn[:, 0:1] + kn[:, 1:2] * kn[:, 1:2] \
        + kn[:, 2:3] * kn[:, 2:3]
    ub = ut.astype(jnp.bfloat16).astype(jnp.float32)
    kb = kn.astype(jnp.bfloat16).astype(jnp.float32)
    g = kb[:, 0:1] * ub[0:1, :] + kb[:, 1:2] * ub[1:2, :] \
        + kb[:, 2:3] * ub[2:3, :]
    d2 = (u2 + k2) - 2.0 * g
    iota = jax.lax.broadcasted_iota(jnp.int32, (m, nt), 0)
    d = d2
    ams, rs = [], []
    for k in range(3):
        mn = jnp.min(d, axis=0, keepdims=True)
        am = jnp.min(jnp.where(d == mn, iota, m), axis=0, keepdims=True)
        ams.append(am)
        rs.append(1.0 / (mn + _EPS_D))
        if k < 2:
            d = jnp.where(iota == am, jnp.float32(1e30), d)
    norm = rs[0] + rs[1] + rs[2]
    wsp = jnp.zeros((m, nt), jnp.float32)
    for k in range(3):
        wsp = jnp.where(iota == ams[k], rs[k] / norm, wsp)
    kf = kf1_ref[0]
    y1 = jnp.dot(kf, wsp.astype(jnp.bfloat16), preferred_element_type=jnp.float32)
    uf = uf_ref[0].astype(jnp.bfloat16)
    y1 = y1 + jnp.dot(w1b_ref[...].astype(jnp.bfloat16), uf,
                      preferred_element_type=jnp.float32)
    y1 = y1 + b1_ref[...]
    step = pl.program_id(0) * pl.num_programs(1) + pl.program_id(1)
    nsteps = pl.num_programs(0) * pl.num_programs(1)

    @pl.when(step == 0)
    def _():
        sum_ref[...] = jnp.zeros_like(sum_ref)
        ssq_ref[...] = jnp.zeros_like(ssq_ref)

    sum_ref[...] += jnp.sum(y1, axis=1, keepdims=True)
    ssq_ref[...] += jnp.sum(y1 * y1, axis=1, keepdims=True)
    y1_ref[0] = y1.astype(jnp.bfloat16)

    @pl.when(step == nsteps - 1)
    def _():
        mu = sum_ref[...] * (1.0 / cnt)
        var = ssq_ref[...] * (1.0 / cnt) - mu * mu
        s = g1_ref[...] * jax.lax.rsqrt(var + _EPS_BN)
        sc1_ref[...] = s
        sh1_ref[...] = be1_ref[...] - s * mu


def _phase_b_kernel(y1_ref, sc1_ref, sh1_ref, w2_ref, g2_ref, be2_ref,
                    sc2_ref, sh2_ref, hs_ref, hh_ref, *, cnt):
    h = jnp.maximum(y1_ref[0].astype(jnp.float32) * sc1_ref[...] + sh1_ref[...],
                    0.0)
    hb = h.astype(jnp.bfloat16)
    b = pl.program_id(0)

    @pl.when(b == 0)
    def _():
        hs_ref[...] = jnp.zeros_like(hs_ref)
        hh_ref[...] = jnp.zeros_like(hh_ref)

    hs_ref[...] += jnp.sum(h, axis=1, keepdims=True)
    hh_ref[...] += jax.lax.dot_general(hb, hb, (((1,), (1,)), ((), ())),
                                       preferred_element_type=jnp.float32)

    @pl.when(b == pl.num_programs(0) - 1)
    def _():
        w2 = w2_ref[...]
        muh = hs_ref[...] * (1.0 / cnt)
        e2 = hh_ref[...] * (1.0 / cnt)
        p = jax.lax.dot_general(w2, muh, (((1,), (0,)), ((), ())),
                                precision=jax.lax.Precision.HIGHEST,
                                preferred_element_type=jnp.float32)
        a = jax.lax.dot_general(w2, e2, (((1,), (0,)), ((), ())),
                                precision=jax.lax.Precision.HIGHEST,
                                preferred_element_type=jnp.float32)
        q = jnp.sum(a * w2, axis=1, keepdims=True)
        var2 = q - p * p
        s2 = g2_ref[...] * jax.lax.rsqrt(var2 + _EPS_BN)
        sc2_ref[...] = s2
        sh2_ref[...] = be2_ref[...] - s2 * p


def _phase_c_kernel(y1_ref, sc1_ref, sh1_ref, w2_ref, sc2_ref, sh2_ref,
                    out_ref):
    h = jnp.maximum(y1_ref[0].astype(jnp.float32) * sc1_ref[...] + sh1_ref[...],
                    0.0)
    z = jnp.dot(w2_ref[...].astype(jnp.bfloat16), h.astype(jnp.bfloat16),
                preferred_element_type=jnp.float32)
    out_ref[0] = jnp.maximum(sc2_ref[...] * z + sh2_ref[...], 0.0)


def kernel(unknown, known, unknow_feats, known_feats, W1, b1, g1, be1, W2, b2,
           g2, be2):
    B, N, _ = unknown.shape
    M = known.shape[1]
    C1 = unknow_feats.shape[1]
    C2 = known_feats.shape[1]
    C0 = W1.shape[0]
    cnt = float(B * N)
    NT_A = 512
    NT_C = 2048

    w1a = W1[:, :C2]
    w1b = W1[:, C2:]
    ut = jnp.swapaxes(unknown, 1, 2)
    col = lambda v: v[:, None]

    kf1 = pl.pallas_call(
        _kf1_kernel,
        grid=(B,),
        in_specs=[
            pl.BlockSpec((C0, C2), lambda b: (0, 0)),
            pl.BlockSpec((1, C2, M), lambda b: (b, 0, 0)),
        ],
        out_specs=pl.BlockSpec((1, C0, M), lambda b: (b, 0, 0)),
        out_shape=jax.ShapeDtypeStruct((B, C0, M), jnp.bfloat16),
    )(w1a, known_feats)

    y1, sc1, sh1 = pl.pallas_call(
        functools.partial(_phase_a_kernel, nt=NT_A, m=M, cnt=cnt),
        grid=(B, N // NT_A),
        in_specs=[
            pl.BlockSpec((1, 3, NT_A), lambda b, j: (b, 0, j)),
            pl.BlockSpec((1, M, 3), lambda b, j: (b, 0, 0)),
            pl.BlockSpec((1, C0, M), lambda b, j: (b, 0, 0)),
            pl.BlockSpec((1, C1, NT_A), lambda b, j: (b, 0, j)),
            pl.BlockSpec((C0, C1), lambda b, j: (0, 0)),
            pl.BlockSpec((C0, 1), lambda b, j: (0, 0)),
            pl.BlockSpec((C0, 1), lambda b, j: (0, 0)),
            pl.BlockSpec((C0, 1), lambda b, j: (0, 0)),
        ],
        out_specs=[
            pl.BlockSpec((1, C0, NT_A), lambda b, j: (b, 0, j)),
            pl.BlockSpec((C0, 1), lambda b, j: (0, 0)),
            pl.BlockSpec((C0, 1), lambda b, j: (0, 0)),
        ],
        out_shape=[
            jax.ShapeDtypeStruct((B, C0, N), jnp.bfloat16),
            jax.ShapeDtypeStruct((C0, 1), jnp.float32),
            jax.ShapeDtypeStruct((C0, 1), jnp.float32),
        ],
        scratch_shapes=[
            pltpu.VMEM((C0, 1), jnp.float32),
            pltpu.VMEM((C0, 1), jnp.float32),
        ],
    )(ut, known, kf1, unknow_feats, w1b, col(b1), col(g1), col(be1))

    sc2, sh2 = pl.pallas_call(
        functools.partial(_phase_b_kernel, cnt=cnt),
        grid=(B,),
        in_specs=[
            pl.BlockSpec((1, C0, N), lambda b: (b, 0, 0)),
            pl.BlockSpec((C0, 1), lambda b: (0, 0)),
            pl.BlockSpec((C0, 1), lambda b: (0, 0)),
            pl.BlockSpec((C0, C0), lambda b: (0, 0)),
            pl.BlockSpec((C0, 1), lambda b: (0, 0)),
            pl.BlockSpec((C0, 1), lambda b: (0, 0)),
        ],
        out_specs=[
            pl.BlockSpec((C0, 1), lambda b: (0, 0)),
            pl.BlockSpec((C0, 1), lambda b: (0, 0)),
        ],
        out_shape=[
            jax.ShapeDtypeStruct((C0, 1), jnp.float32),
            jax.ShapeDtypeStruct((C0, 1), jnp.float32),
        ],
        scratch_shapes=[
            pltpu.VMEM((C0, 1), jnp.float32),
            pltpu.VMEM((C0, C0), jnp.float32),
        ],
    )(y1, sc1, sh1, W2, col(g2), col(be2))

    out = pl.pallas_call(
        _phase_c_kernel,
        grid=(B, N // NT_C),
        in_specs=[
            pl.BlockSpec((1, C0, NT_C), lambda b, j: (b, 0, j)),
            pl.BlockSpec((C0, 1), lambda b, j: (0, 0)),
            pl.BlockSpec((C0, 1), lambda b, j: (0, 0)),
            pl.BlockSpec((C0, C0), lambda b, j: (0, 0)),
            pl.BlockSpec((C0, 1), lambda b, j: (0, 0)),
            pl.BlockSpec((C0, 1), lambda b, j: (0, 0)),
        ],
        out_specs=pl.BlockSpec((1, C0, NT_C), lambda b, j: (b, 0, j)),
        out_shape=jax.ShapeDtypeStruct((B, C0, N), jnp.float32),
    )(y1, sc1, sh1, W2, sc2, sh2)
    return out
